# Initial kernel scaffold; baseline (speedup 1.0000x reference)
import jax
import jax.numpy as jnp
from jax.experimental import pallas as pl


def kernel(x, W1, b1, W2, b2, Wg, bg):
    raise NotImplementedError("write your pallas kernel here")


# speedup vs baseline: 7.6722x; 7.6722x over previous
import jax
import jax.numpy as jnp
from jax.experimental import pallas as pl
from jax.experimental.pallas import tpu as pltpu

_NE = 8
_D = 1024
_H = 4096
_BH = 512
_BM = 1024


def _gelu_exact(v):
    return 0.5 * v * (1.0 + jax.lax.erf(v * 0.7071067811865476))


def _moe_kernel(x_ref, Wg_ref, bg_ref, W1_ref, b1_ref, W2_ref, b2_ref,
                out_ref, mask_ref):
    e = pl.program_id(1)
    h = pl.program_id(2)

    @pl.when((e == 0) & (h == 0))
    def _init():
        scores = jnp.dot(x_ref[...], Wg_ref[...],
                         preferred_element_type=jnp.float32) + bg_ref[...]
        lane = jax.lax.broadcasted_iota(jnp.int32, scores.shape, 1)
        m1 = jnp.max(scores, axis=-1, keepdims=True)
        first1 = jnp.min(jnp.where(scores == m1, lane, _NE), axis=-1,
                         keepdims=True)
        rest = jnp.where(lane == first1, -jnp.inf, scores)
        m2 = jnp.max(rest, axis=-1, keepdims=True)
        first2 = jnp.min(jnp.where(rest == m2, lane, _NE), axis=-1,
                         keepdims=True)
        mask_ref[...] = ((lane == first1) | (lane == first2)).astype(
            jnp.float32)
        out_ref[...] = jnp.zeros_like(out_ref)

    lane = jax.lax.broadcasted_iota(jnp.int32, mask_ref.shape, 1)
    mcol = jnp.sum(jnp.where(lane == e, mask_ref[...], 0.0), axis=-1,
                   keepdims=True)

    hblk = jnp.dot(x_ref[...], W1_ref[0],
                   preferred_element_type=jnp.float32) + b1_ref[0]
    hblk = _gelu_exact(hblk) * mcol
    out_ref[...] += jnp.dot(hblk, W2_ref[0],
                            preferred_element_type=jnp.float32)

    @pl.when(h == 0)
    def _bias2():
        out_ref[...] += mcol * b2_ref[0]


def kernel(x, W1, b1, W2, b2, Wg, bg):
    B, S, D = x.shape
    T = B * S
    x2 = x.reshape(T, D)
    bg2 = bg.reshape(1, _NE)
    b1_3 = b1.reshape(_NE, 1, _H)
    b2_3 = b2.reshape(_NE, 1, D)

    nh = _H // _BH
    nm = T // _BM
    out = pl.pallas_call(
        _moe_kernel,
        grid=(nm, _NE, nh),
        in_specs=[
            pl.BlockSpec((_BM, D), lambda m, e, h: (m, 0)),
            pl.BlockSpec((D, _NE), lambda m, e, h: (0, 0)),
            pl.BlockSpec((1, _NE), lambda m, e, h: (0, 0)),
            pl.BlockSpec((1, D, _BH), lambda m, e, h: (e, 0, h)),
            pl.BlockSpec((1, 1, _BH), lambda m, e, h: (e, 0, h)),
            pl.BlockSpec((1, _BH, D), lambda m, e, h: (e, h, 0)),
            pl.BlockSpec((1, 1, D), lambda m, e, h: (e, 0, 0)),
        ],
        out_specs=pl.BlockSpec((_BM, D), lambda m, e, h: (m, 0)),
        out_shape=jax.ShapeDtypeStruct((T, D), jnp.float32),
        scratch_shapes=[pltpu.VMEM((_BM, _NE), jnp.float32)],
    )(x2, Wg, bg2, W1, b1_3, W2, b2_3)
    return out.reshape(B, S, D)
